# baseline (speedup 1.0000x reference)
import jax
import jax.numpy as jnp
from jax.experimental import pallas as pl
from jax.experimental.pallas import tpu as pltpu

N = 100000
D_IN = 128
D_OUT = 128
BM = 2000
NBUF = 8
NBLK = N // BM


def _gconv_body(x_hbm, w_ref, a_ref, o_hbm,
                in_buf, out_buf, c_ref, in_sems, out_sems):
    i = pl.program_id(0)
    s = jax.lax.rem(i, NBUF)

    def in_copy(blk, slot):
        return pltpu.make_async_copy(
            x_hbm.at[pl.ds(blk * BM, BM), :], in_buf.at[slot],
            in_sems.at[slot])

    def out_copy(blk, slot):
        return pltpu.make_async_copy(
            out_buf.at[slot], o_hbm.at[pl.ds(blk * BM, BM), :],
            out_sems.at[slot])

    @pl.when(i == 0)
    def _():
        c = jnp.dot(w_ref[...], a_ref[...], preferred_element_type=jnp.float32)
        c_ref[...] = c.astype(jnp.bfloat16)
        for k in range(NBUF):
            in_copy(k, k).start()

    in_copy(i, s).wait()

    @pl.when(i >= NBUF)
    def _():
        out_copy(i - NBUF, s).wait()

    out_buf[s] = in_buf[s]
    out_copy(i, s).start()

    @pl.when(i + NBUF < NBLK)
    def _():
        in_copy(i + NBUF, s).start()

    @pl.when(i == NBLK - 1)
    def _():
        for k in range(NBUF):
            blk = NBLK - NBUF + k
            out_copy(blk, jax.lax.rem(jnp.int32(blk), NBUF)).wait()


@jax.jit
def kernel(x, W, adj):
    return pl.pallas_call(
        _gconv_body,
        grid=(NBLK,),
        in_specs=[
            pl.BlockSpec(memory_space=pl.ANY),
            pl.BlockSpec((D_IN, D_OUT), lambda i: (0, 0)),
            pl.BlockSpec((D_OUT, D_OUT), lambda i: (0, 0)),
        ],
        out_specs=pl.BlockSpec(memory_space=pl.ANY),
        out_shape=jax.ShapeDtypeStruct((N, D_OUT), jnp.float32),
        scratch_shapes=[
            pltpu.VMEM((NBUF, BM, D_IN), jnp.float32),
            pltpu.VMEM((NBUF, BM, D_OUT), jnp.float32),
            pltpu.VMEM((D_IN, D_OUT), jnp.bfloat16),
            pltpu.SemaphoreType.DMA((NBUF,)),
            pltpu.SemaphoreType.DMA((NBUF,)),
        ],
        compiler_params=pltpu.CompilerParams(
            dimension_semantics=("arbitrary",),
        ),
    )(x, W, adj)


# speedup vs baseline: 1.0047x; 1.0047x over previous
import jax
import jax.numpy as jnp
from jax.experimental import pallas as pl
from jax.experimental.pallas import tpu as pltpu

N = 100000
D_IN = 128
D_OUT = 128
BM = 4000
NBUF = 6
NBLK = N // BM


def _gconv_body(x_hbm, w_ref, a_ref, o_hbm,
                in_buf, out_buf, c_ref, in_sems, out_sems):
    i = pl.program_id(0)
    s = jax.lax.rem(i, NBUF)

    def in_copy(blk, slot):
        return pltpu.make_async_copy(
            x_hbm.at[pl.ds(blk * BM, BM), :], in_buf.at[slot],
            in_sems.at[slot])

    def out_copy(blk, slot):
        return pltpu.make_async_copy(
            out_buf.at[slot], o_hbm.at[pl.ds(blk * BM, BM), :],
            out_sems.at[slot])

    @pl.when(i == 0)
    def _():
        c = jnp.dot(w_ref[...], a_ref[...], preferred_element_type=jnp.float32)
        c_ref[...] = c.astype(jnp.bfloat16)
        for k in range(NBUF):
            in_copy(k, k).start()

    in_copy(i, s).wait()

    @pl.when(i >= NBUF)
    def _():
        out_copy(i - NBUF, s).wait()

    out_buf[s] = jnp.dot(in_buf[s].astype(jnp.bfloat16), c_ref[...],
                         preferred_element_type=jnp.float32)
    out_copy(i, s).start()

    @pl.when(i + NBUF < NBLK)
    def _():
        in_copy(i + NBUF, s).start()

    @pl.when(i == NBLK - 1)
    def _():
        for k in range(NBUF):
            blk = NBLK - NBUF + k
            out_copy(blk, jax.lax.rem(jnp.int32(blk), NBUF)).wait()


@jax.jit
def kernel(x, W, adj):
    return pl.pallas_call(
        _gconv_body,
        grid=(NBLK,),
        in_specs=[
            pl.BlockSpec(memory_space=pl.ANY),
            pl.BlockSpec((D_IN, D_OUT), lambda i: (0, 0)),
            pl.BlockSpec((D_OUT, D_OUT), lambda i: (0, 0)),
        ],
        out_specs=pl.BlockSpec(memory_space=pl.ANY),
        out_shape=jax.ShapeDtypeStruct((N, D_OUT), jnp.float32),
        scratch_shapes=[
            pltpu.VMEM((NBUF, BM, D_IN), jnp.float32),
            pltpu.VMEM((NBUF, BM, D_OUT), jnp.float32),
            pltpu.VMEM((D_IN, D_OUT), jnp.bfloat16),
            pltpu.SemaphoreType.DMA((NBUF,)),
            pltpu.SemaphoreType.DMA((NBUF,)),
        ],
        compiler_params=pltpu.CompilerParams(
            dimension_semantics=("arbitrary",),
        ),
    )(x, W, adj)
